# baseline (speedup 1.0000x reference)
import functools

import jax
import jax.numpy as jnp
from jax import lax
from jax.experimental import pallas as pl
from jax.experimental.pallas import tpu as pltpu
from jax.experimental.pallas import tpu_sc as plsc

_VOCAB = 1000000
_D = 64
_SCALE = float(_D) ** 0.5

_NC = 2
_NS = 16
_NW = _NC * _NS

_B = 16384 * 50
_C = 128
_PER_W = _B // _NW
_NCHUNK = _PER_W // _C


def _sc_body(table_hbm, idx_hbm, out_hbm, idx_v, rows_v, gsem):
    c = lax.axis_index("c")
    s = lax.axis_index("s")
    wid = s * _NC + c
    base_chunk = wid * _NCHUNK

    pltpu.sync_copy(idx_hbm.at[pl.ds(base_chunk, _NCHUNK)], idx_v)

    def chunk(j, carry):
        pltpu.async_copy(table_hbm.at[idx_v.at[j]], rows_v, gsem).wait()

        def row(i, carry2):
            for u in range(_D // 16):
                sl = pl.ds(u * 16, 16)
                rows_v[i, sl] = rows_v[i, sl] * _SCALE
            return carry2

        lax.fori_loop(0, _C, row, 0)

        pltpu.sync_copy(rows_v, out_hbm.at[pl.ds((base_chunk + j) * _C, _C)])
        return carry

    lax.fori_loop(0, _NCHUNK, chunk, 0)


@jax.jit
def _embed(table, idx2d):
    mesh = plsc.VectorSubcoreMesh(core_axis_name="c", subcore_axis_name="s")
    k = functools.partial(
        pl.kernel,
        out_type=jax.ShapeDtypeStruct((_B, _D), jnp.float32),
        mesh=mesh,
        scratch_types=[
            pltpu.VMEM((_NCHUNK, _C), jnp.int32),
            pltpu.VMEM((_C, _D), jnp.float32),
            pltpu.SemaphoreType.DMA,
        ],
        compiler_params=pltpu.CompilerParams(use_tc_tiling_on_sc=False),
    )(_sc_body)
    return k(table, idx2d)


def kernel(xb, table):
    idx2d = xb.astype(jnp.int32).reshape(_B // _C, _C)
    out = _embed(table, idx2d)
    return out.reshape(xb.shape + (_D,))


# speedup vs baseline: 1.2048x; 1.2048x over previous
import functools

import jax
import jax.numpy as jnp
from jax import lax
from jax.experimental import pallas as pl
from jax.experimental.pallas import tpu as pltpu
from jax.experimental.pallas import tpu_sc as plsc

_VOCAB = 1000000
_D = 64
_SCALE = float(_D) ** 0.5

_NC = 2
_NS = 16
_NW = _NC * _NS

_B = 16384 * 50
_C = 128
_K = 2
_S = _C * _K
_PER_W = _B // _NW
_NSUP = _PER_W // _S
_NCHUNK = _PER_W // _C
_NBUF = 5
_LOOK = 2


def _sc_body(table_hbm, idx_hbm, out_hbm, idx_v, rows_v, *sems):
    gsems = sems[:_NBUF]
    osems = sems[_NBUF:]

    c = lax.axis_index("c")
    s = lax.axis_index("s")
    wid = s * _NC + c
    base_chunk = wid * _NCHUNK
    base_row = wid * _PER_W

    pltpu.sync_copy(idx_hbm.at[pl.ds(base_chunk, _NCHUNK)], idx_v)

    def issue_gathers(q, r):
        for t in range(_K):
            d = q * _K + t
            pltpu.async_copy(
                table_hbm.at[idx_v.at[d]],
                rows_v.at[r].at[pl.ds(t * _C, _C)],
                gsems[r],
            )

    def drain_gathers(r):
        pltpu.make_async_copy(
            table_hbm.at[pl.ds(0, _S)], rows_v.at[r], gsems[r]
        ).wait()

    def scale(r):
        @plsc.parallel_loop(0, _S, 1, unroll=4)
        def _(i):
            for u in range(_D // 16):
                sl = pl.ds(u * 16, 16)
                rows_v[r, i, sl] = rows_v[r, i, sl] * _SCALE

    def issue_out(q, r):
        pltpu.async_copy(
            rows_v.at[r],
            out_hbm.at[pl.ds(base_row + q * _S, _S)],
            osems[r],
        )

    def drain_out(r):
        pltpu.make_async_copy(
            rows_v.at[r], out_hbm.at[pl.ds(0, _S)], osems[r]
        ).wait()

    issue_gathers(jnp.int32(0), 0)
    issue_gathers(jnp.int32(1), 1)
    for r in range(_NBUF):
        q = jnp.int32(r)
        r2 = (r + _LOOK) % _NBUF
        if r + _LOOK >= _NBUF:
            drain_out(r2)
        issue_gathers(q + _LOOK, r2)
        drain_gathers(r)
        scale(r)
        issue_out(q, r)

    def outer(o, carry):
        for r in range(_NBUF):
            q = o * _NBUF + r
            r2 = (r + _LOOK) % _NBUF
            drain_out(r2)
            issue_gathers(q + _LOOK, r2)
            drain_gathers(r)
            scale(r)
            issue_out(q, r)
        return carry

    lax.fori_loop(1, _NSUP // _NBUF - 1, outer, 0)

    for r in range(_NBUF):
        q = jnp.int32(_NSUP - _NBUF + r)
        r2 = (r + _LOOK) % _NBUF
        if r + _LOOK < _NBUF:
            drain_out(r2)
            issue_gathers(q + _LOOK, r2)
        drain_gathers(r)
        scale(r)
        issue_out(q, r)
    for r in range(_NBUF):
        drain_out(r)


@jax.jit
def _embed(table, idx2d):
    mesh = plsc.VectorSubcoreMesh(core_axis_name="c", subcore_axis_name="s")
    k = functools.partial(
        pl.kernel,
        out_type=jax.ShapeDtypeStruct((_B, _D), jnp.float32),
        mesh=mesh,
        scratch_types=[
            pltpu.VMEM((_NCHUNK, _C), jnp.int32),
            pltpu.VMEM((_NBUF, _S, _D), jnp.float32),
        ]
        + [pltpu.SemaphoreType.DMA] * (2 * _NBUF),
        compiler_params=pltpu.CompilerParams(use_tc_tiling_on_sc=False),
    )(_sc_body)
    return k(table, idx2d)


def kernel(xb, table):
    idx2d = xb.astype(jnp.int32).reshape(_B // _C, _C)
    out = _embed(table, idx2d)
    return out.reshape(xb.shape + (_D,))
